# baseline (speedup 1.0000x reference)
import functools

import jax
import jax.numpy as jnp
from jax import lax
import jax.experimental.pallas as pl
from jax.experimental.pallas import tpu as pltpu
from jax.experimental.pallas import tpu_sc as plsc

DEG = 30.0
B, H, W, C = 4, 384, 384, 96
N = B * H * W

NC, NS, L = 2, 16, 16
NW = NC * NS
ROWS_PER_W = N // NW
G = 128
NG = 4
CHUNK = G * NG
NCHUNK = ROWS_PER_W // CHUNK


def _index_kernel(cos_ref, sin_ref, idx_ref, mask_ref):
    b = pl.program_id(0)
    off_y = (H - 1) / 2.0
    off_x = (W - 1) / 2.0

    def bf(v):
        return v.astype(jnp.bfloat16).astype(jnp.float32)

    c = bf(jnp.full((H, W), cos_ref[b], jnp.float32))
    s = bf(jnp.full((H, W), sin_ref[b], jnp.float32))
    i = bf(lax.broadcasted_iota(jnp.int32, (H, W), 0).astype(jnp.float32) - off_y)
    j = bf(lax.broadcasted_iota(jnp.int32, (H, W), 1).astype(jnp.float32) - off_x)
    y = jnp.round(c * i - s * j + off_y)
    x = jnp.round(s * i + c * j + off_x)
    oor = (y < 0) | (y >= H) | (x < 0) | (x >= W)
    yc = jnp.clip(y, 0.0, float(H - 1)).astype(jnp.int32)
    xc = jnp.clip(x, 0.0, float(W - 1)).astype(jnp.int32)
    idx = b * (H * W) + yc * W + xc
    idx_ref[0] = idx
    mask_ref[0] = jnp.where(oor, 0.0, 1.0).astype(jnp.float32)


def _compute_indices(cos_v, sin_v):
    return pl.pallas_call(
        _index_kernel,
        grid=(B,),
        in_specs=[
            pl.BlockSpec(memory_space=pltpu.SMEM),
            pl.BlockSpec(memory_space=pltpu.SMEM),
        ],
        out_specs=[
            pl.BlockSpec((1, H, W), lambda b: (b, 0, 0)),
            pl.BlockSpec((1, H, W), lambda b: (b, 0, 0)),
        ],
        out_shape=[
            jax.ShapeDtypeStruct((B, H, W), jnp.int32),
            jax.ShapeDtypeStruct((B, H, W), jnp.float32),
        ],
    )(cos_v, sin_v)


def _sc_gather(table, idx, mask):
    mesh = plsc.VectorSubcoreMesh(
        core_axis_name="c", subcore_axis_name="s", num_cores=NC, num_subcores=NS
    )

    @functools.partial(
        pl.kernel,
        mesh=mesh,
        out_type=jax.ShapeDtypeStruct((N, C), jnp.float32),
        compiler_params=pltpu.CompilerParams(use_tc_tiling_on_sc=False),
        scratch_types=[
            pltpu.VMEM((CHUNK,), jnp.int32),
            pltpu.VMEM((CHUNK,), jnp.float32),
            pltpu.VMEM((CHUNK, C), jnp.float32),
            pltpu.SemaphoreType.DMA,
        ],
    )
    def k(table_hbm, idx_hbm, mask_hbm, out_hbm, idx_v, mask_v, rows_v, sem):
        wid = lax.axis_index("s") * NC + lax.axis_index("c")
        base = wid * ROWS_PER_W

        def chunk_body(kk, carry):
            off = base + kk * CHUNK
            pltpu.sync_copy(idx_hbm.at[pl.ds(off, CHUNK)], idx_v)
            pltpu.sync_copy(mask_hbm.at[pl.ds(off, CHUNK)], mask_v)
            for g in range(NG):
                pltpu.async_copy(
                    table_hbm.at[idx_v.at[pl.ds(g * G, G)]],
                    rows_v.at[pl.ds(g * G, G)],
                    sem,
                )
            for g in range(NG):
                pltpu.make_async_copy(
                    table_hbm.at[idx_v.at[pl.ds(g * G, G)]],
                    rows_v.at[pl.ds(g * G, G)],
                    sem,
                ).wait()

            def group_body(gr, c2):
                m16 = mask_v[pl.ds(gr * L, L)]
                for l in range(L):
                    r = gr * L + l
                    mv = jnp.broadcast_to(m16[l], (L,))
                    for cc in range(C // L):
                        sl = pl.ds(cc * L, L)
                        rows_v[r, sl] = rows_v[r, sl] * mv
                return c2

            lax.fori_loop(0, CHUNK // L, group_body, 0)
            pltpu.sync_copy(rows_v, out_hbm.at[pl.ds(off, CHUNK)])
            return carry

        lax.fori_loop(0, NCHUNK, chunk_body, 0)

    return k(table, idx, mask)


def kernel(data, angle_u):
    angle = jnp.deg2rad(angle_u * (2 * DEG) - DEG)
    cos_v = jnp.cos(angle)
    sin_v = jnp.sin(angle)
    idx, mask = _compute_indices(cos_v, sin_v)
    table = data.reshape(N, C)
    out = _sc_gather(table, idx.reshape(N), mask.reshape(N))
    return out.reshape(B, H, W, C)


# speedup vs baseline: 1.0556x; 1.0556x over previous
import functools

import jax
import jax.numpy as jnp
from jax import lax
import jax.experimental.pallas as pl
from jax.experimental.pallas import tpu as pltpu
from jax.experimental.pallas import tpu_sc as plsc

DEG = 30.0
B, H, W, C = 4, 384, 384, 96
N = B * H * W

NC, NS, L = 2, 16, 16
NW = NC * NS
ROWS_PER_W = N // NW
G = 128
NG = 4
CHUNK = G * NG
NCHUNK = ROWS_PER_W // CHUNK


def _index_kernel(cos_ref, sin_ref, idx_ref, mask_ref):
    b = pl.program_id(0)
    off_y = (H - 1) / 2.0
    off_x = (W - 1) / 2.0

    def bf(v):
        return v.astype(jnp.bfloat16).astype(jnp.float32)

    c = bf(jnp.full((H, W), cos_ref[b], jnp.float32))
    s = bf(jnp.full((H, W), sin_ref[b], jnp.float32))
    i = bf(lax.broadcasted_iota(jnp.int32, (H, W), 0).astype(jnp.float32) - off_y)
    j = bf(lax.broadcasted_iota(jnp.int32, (H, W), 1).astype(jnp.float32) - off_x)
    y = jnp.round(c * i - s * j + off_y)
    x = jnp.round(s * i + c * j + off_x)
    oor = (y < 0) | (y >= H) | (x < 0) | (x >= W)
    yc = jnp.clip(y, 0.0, float(H - 1)).astype(jnp.int32)
    xc = jnp.clip(x, 0.0, float(W - 1)).astype(jnp.int32)
    idx = b * (H * W) + yc * W + xc
    idx_ref[0] = idx
    mask_ref[0] = jnp.where(oor, 0.0, 1.0).astype(jnp.float32)


def _compute_indices(cos_v, sin_v):
    return pl.pallas_call(
        _index_kernel,
        grid=(B,),
        in_specs=[
            pl.BlockSpec(memory_space=pltpu.SMEM),
            pl.BlockSpec(memory_space=pltpu.SMEM),
        ],
        out_specs=[
            pl.BlockSpec((1, H, W), lambda b: (b, 0, 0)),
            pl.BlockSpec((1, H, W), lambda b: (b, 0, 0)),
        ],
        out_shape=[
            jax.ShapeDtypeStruct((B, H, W), jnp.int32),
            jax.ShapeDtypeStruct((B, H, W), jnp.float32),
        ],
    )(cos_v, sin_v)


def _sc_gather(table, idx, mask):
    mesh = plsc.VectorSubcoreMesh(
        core_axis_name="c", subcore_axis_name="s", num_cores=NC, num_subcores=NS
    )

    @functools.partial(
        pl.kernel,
        mesh=mesh,
        out_type=jax.ShapeDtypeStruct((N, C), jnp.float32),
        compiler_params=pltpu.CompilerParams(use_tc_tiling_on_sc=False),
        scratch_types=[
            pltpu.VMEM((2, CHUNK), jnp.int32),
            pltpu.VMEM((2, CHUNK), jnp.float32),
            pltpu.VMEM((2 * CHUNK, C), jnp.float32),
            pltpu.SemaphoreType.DMA,
            pltpu.SemaphoreType.DMA,
            pltpu.SemaphoreType.DMA,
            pltpu.SemaphoreType.DMA,
        ],
    )
    def k(table_hbm, idx_hbm, mask_hbm, out_hbm, idx_v, mask_v, rows_v,
          sem_i0, sem_i1, sem_g, sem_o):
        wid = lax.axis_index("s") * NC + lax.axis_index("c")
        base = wid * ROWS_PER_W
        sem_i = (sem_i0, sem_i1)

        def start_in(kk, buf):
            off = base + kk * CHUNK
            pltpu.async_copy(idx_hbm.at[pl.ds(off, CHUNK)], idx_v.at[buf], sem_i[buf])
            pltpu.async_copy(mask_hbm.at[pl.ds(off, CHUNK)], mask_v.at[buf], sem_i[buf])

        def wait_in(kk, buf):
            off = base + kk * CHUNK
            pltpu.make_async_copy(idx_hbm.at[pl.ds(off, CHUNK)], idx_v.at[buf], sem_i[buf]).wait()
            pltpu.make_async_copy(mask_hbm.at[pl.ds(off, CHUNK)], mask_v.at[buf], sem_i[buf]).wait()

        def start_gather(kk, buf):
            for g in range(NG):
                pltpu.async_copy(
                    table_hbm.at[idx_v.at[buf, pl.ds(g * G, G)]],
                    rows_v.at[pl.ds(buf * CHUNK + g * G, G)],
                    sem_g,
                )

        def wait_gather(kk, buf):
            for g in range(NG):
                pltpu.make_async_copy(
                    table_hbm.at[idx_v.at[buf, pl.ds(g * G, G)]],
                    rows_v.at[pl.ds(buf * CHUNK + g * G, G)],
                    sem_g,
                ).wait()

        def mask_mul(buf):
            def group_body(gr, c2):
                m16 = mask_v[buf, pl.ds(gr * L, L)]
                for l in range(L):
                    r = buf * CHUNK + gr * L + l
                    mv = jnp.broadcast_to(m16[l], (L,))
                    for cc in range(C // L):
                        sl = pl.ds(cc * L, L)
                        rows_v[r, sl] = rows_v[r, sl] * mv
                return c2

            lax.fori_loop(0, CHUNK // L, group_body, 0)

        def start_out(kk, buf):
            off = base + kk * CHUNK
            pltpu.async_copy(
                rows_v.at[pl.ds(buf * CHUNK, CHUNK)], out_hbm.at[pl.ds(off, CHUNK)], sem_o
            )

        def wait_out(kk, buf):
            off = base + kk * CHUNK
            pltpu.make_async_copy(
                rows_v.at[pl.ds(buf * CHUNK, CHUNK)], out_hbm.at[pl.ds(off, CHUNK)], sem_o
            ).wait()

        start_in(0, 0)
        wait_in(0, 0)
        start_gather(0, 0)
        start_in(1, 1)
        NSTEP = NCHUNK // 2

        def step_body(t, carry):
            k0 = 2 * t
            k1 = k0 + 1
            wait_gather(k0, 0)
            mask_mul(0)
            start_out(k0, 0)
            wait_in(k1, 1)

            @pl.when(t >= 1)
            def _():
                wait_out(k0 - 1, 1)

            start_gather(k1, 1)

            @pl.when(t < NSTEP - 1)
            def _():
                start_in(k0 + 2, 0)

            wait_gather(k1, 1)
            mask_mul(1)
            start_out(k1, 1)

            @pl.when(t < NSTEP - 1)
            def _():
                wait_in(k0 + 2, 0)
                wait_out(k0, 0)
                start_gather(k0 + 2, 0)
                start_in(k0 + 3, 1)

            return carry

        lax.fori_loop(0, NSTEP, step_body, 0)
        wait_out(NCHUNK - 2, 0)
        wait_out(NCHUNK - 1, 1)

    return k(table, idx, mask)


def kernel(data, angle_u):
    angle = jnp.deg2rad(angle_u * (2 * DEG) - DEG)
    cos_v = jnp.cos(angle)
    sin_v = jnp.sin(angle)
    idx, mask = _compute_indices(cos_v, sin_v)
    table = data.reshape(N, C)
    out = _sc_gather(table, idx.reshape(N), mask.reshape(N))
    return out.reshape(B, H, W, C)


# speedup vs baseline: 1.3340x; 1.2638x over previous
import functools

import jax
import jax.numpy as jnp
from jax import lax
import jax.experimental.pallas as pl
from jax.experimental.pallas import tpu as pltpu
from jax.experimental.pallas import tpu_sc as plsc

DEG = 30.0
B, H, W, C = 4, 384, 384, 96
N = B * H * W

NC, NS, L = 2, 16, 16
NW = NC * NS
ROWS_PER_W = N // NW
CP = 128
G = 128
NG = 1
CHUNK = G * NG
NCHUNK = ROWS_PER_W // CHUNK


def _index_kernel(cos_ref, sin_ref, idx_ref, mask_ref):
    b = pl.program_id(0)
    off_y = (H - 1) / 2.0
    off_x = (W - 1) / 2.0

    def bf(v):
        return v.astype(jnp.bfloat16).astype(jnp.float32)

    c = bf(jnp.full((H, W), cos_ref[b], jnp.float32))
    s = bf(jnp.full((H, W), sin_ref[b], jnp.float32))
    i = bf(lax.broadcasted_iota(jnp.int32, (H, W), 0).astype(jnp.float32) - off_y)
    j = bf(lax.broadcasted_iota(jnp.int32, (H, W), 1).astype(jnp.float32) - off_x)
    y = jnp.round(c * i - s * j + off_y)
    x = jnp.round(s * i + c * j + off_x)
    oor = (y < 0) | (y >= H) | (x < 0) | (x >= W)
    yc = jnp.clip(y, 0.0, float(H - 1)).astype(jnp.int32)
    xc = jnp.clip(x, 0.0, float(W - 1)).astype(jnp.int32)
    idx = b * (H * W) + yc * W + xc
    idx_ref[0] = idx
    mask_ref[0] = jnp.where(oor, 0.0, 1.0).astype(jnp.float32)


def _compute_indices(cos_v, sin_v):
    return pl.pallas_call(
        _index_kernel,
        grid=(B,),
        in_specs=[
            pl.BlockSpec(memory_space=pltpu.SMEM),
            pl.BlockSpec(memory_space=pltpu.SMEM),
        ],
        out_specs=[
            pl.BlockSpec((1, H, W), lambda b: (b, 0, 0)),
            pl.BlockSpec((1, H, W), lambda b: (b, 0, 0)),
        ],
        out_shape=[
            jax.ShapeDtypeStruct((B, H, W), jnp.int32),
            jax.ShapeDtypeStruct((B, H, W), jnp.float32),
        ],
    )(cos_v, sin_v)


def _sc_gather(table, idx, mask):
    mesh = plsc.VectorSubcoreMesh(
        core_axis_name="c", subcore_axis_name="s", num_cores=NC, num_subcores=NS
    )

    @functools.partial(
        pl.kernel,
        mesh=mesh,
        out_type=jax.ShapeDtypeStruct((N, C), jnp.float32),
        compiler_params=pltpu.CompilerParams(use_tc_tiling_on_sc=True),
        scratch_types=[
            pltpu.VMEM((CHUNK,), jnp.int32),
            pltpu.VMEM((CHUNK,), jnp.int32),
            pltpu.VMEM((CHUNK,), jnp.float32),
            pltpu.VMEM((CHUNK,), jnp.float32),
            pltpu.VMEM((2 * CHUNK, CP), jnp.float32),
            pltpu.VMEM((2 * CHUNK, C), jnp.float32),
            pltpu.SemaphoreType.DMA,
            pltpu.SemaphoreType.DMA,
            pltpu.SemaphoreType.DMA,
            pltpu.SemaphoreType.DMA,
        ],
    )
    def k(table_hbm, idx_hbm, mask_hbm, out_hbm, idx_v0, idx_v1, mask_v0,
          mask_v1, rows_v, out_v, sem_i0, sem_i1, sem_g, sem_o):
        wid = lax.axis_index("s") * NC + lax.axis_index("c")
        base = wid * ROWS_PER_W
        sem_i = (sem_i0, sem_i1)
        idx_v = (idx_v0, idx_v1)
        mask_v = (mask_v0, mask_v1)

        def start_in(kk, buf):
            off = base + kk * CHUNK
            pltpu.async_copy(idx_hbm.at[pl.ds(off, CHUNK)], idx_v[buf], sem_i[buf])
            pltpu.async_copy(mask_hbm.at[pl.ds(off, CHUNK)], mask_v[buf], sem_i[buf])

        def wait_in(kk, buf):
            off = base + kk * CHUNK
            pltpu.make_async_copy(idx_hbm.at[pl.ds(off, CHUNK)], idx_v[buf], sem_i[buf]).wait()
            pltpu.make_async_copy(mask_hbm.at[pl.ds(off, CHUNK)], mask_v[buf], sem_i[buf]).wait()

        def start_gather(kk, buf):
            for g in range(NG):
                pltpu.async_copy(
                    table_hbm.at[idx_v[buf].at[pl.ds(g * G, G)]],
                    rows_v.at[pl.ds(buf * CHUNK + g * G, G)],
                    sem_g,
                )

        def wait_gather(kk, buf):
            for g in range(NG):
                pltpu.make_async_copy(
                    table_hbm.at[idx_v[buf].at[pl.ds(g * G, G)]],
                    rows_v.at[pl.ds(buf * CHUNK + g * G, G)],
                    sem_g,
                ).wait()

        def mask_mul(buf):
            def group_body(gr, c2):
                m16 = mask_v[buf][pl.ds(gr * L, L)]
                for l in range(L):
                    r = buf * CHUNK + gr * L + l
                    mv = jnp.broadcast_to(m16[l], (L,))
                    for cc in range(C // L):
                        sl = pl.ds(cc * L, L)
                        out_v[r, sl] = rows_v[r, sl] * mv
                return c2

            lax.fori_loop(0, CHUNK // L, group_body, 0)

        def start_out(kk, buf):
            off = base + kk * CHUNK
            pltpu.async_copy(
                out_v.at[pl.ds(buf * CHUNK, CHUNK)],
                out_hbm.at[pl.ds(off, CHUNK)],
                sem_o,
            )

        def wait_out(kk, buf):
            off = base + kk * CHUNK
            pltpu.make_async_copy(
                out_v.at[pl.ds(buf * CHUNK, CHUNK)],
                out_hbm.at[pl.ds(off, CHUNK)],
                sem_o,
            ).wait()

        start_in(0, 0)
        wait_in(0, 0)
        start_gather(0, 0)
        start_in(1, 1)
        NSTEP = NCHUNK // 2

        def step_body(t, carry):
            k0 = 2 * t
            k1 = k0 + 1
            wait_gather(k0, 0)
            mask_mul(0)
            start_out(k0, 0)
            wait_in(k1, 1)

            @pl.when(t >= 1)
            def _():
                wait_out(k0 - 1, 1)

            start_gather(k1, 1)

            @pl.when(t < NSTEP - 1)
            def _():
                start_in(k0 + 2, 0)

            wait_gather(k1, 1)
            mask_mul(1)
            start_out(k1, 1)

            @pl.when(t < NSTEP - 1)
            def _():
                wait_in(k0 + 2, 0)
                wait_out(k0, 0)
                start_gather(k0 + 2, 0)
                start_in(k0 + 3, 1)

            return carry

        lax.fori_loop(0, NSTEP, step_body, 0)
        wait_out(NCHUNK - 2, 0)
        wait_out(NCHUNK - 1, 1)

    return k(table, idx, mask)


def kernel(data, angle_u):
    angle = jnp.deg2rad(angle_u * (2 * DEG) - DEG)
    cos_v = jnp.cos(angle)
    sin_v = jnp.sin(angle)
    idx, mask = _compute_indices(cos_v, sin_v)
    table = jnp.pad(data.reshape(N, C), ((0, 0), (0, CP - C)))
    out = _sc_gather(table, idx.reshape(N), mask.reshape(N))
    return out.reshape(B, H, W, C)
